# baseline (speedup 1.0000x reference)
import functools

import jax
import jax.numpy as jnp
from jax.experimental import pallas as pl
from jax.experimental.pallas import tpu as pltpu
from jax.experimental.pallas import tpu_sc as plsc

_VOCAB_TILE = 4096


def _sc_gather_body(ids_hbm, emb_hbm, x_hbm):
    num_cores = jax.lax.axis_size("c")
    c = jax.lax.axis_index("c")
    s = jax.lax.axis_index("s")
    wid = s * num_cores + c
    n_tok = x_hbm.shape[0]
    n_workers = num_cores * jax.lax.axis_size("s")
    rows = n_tok // n_workers

    def scoped(ids_vmem, tile, sem):
        pltpu.make_async_copy(
            ids_hbm.at[pl.ds(wid * rows, rows)], ids_vmem, sem
        ).start()
        pltpu.make_async_copy(
            ids_hbm.at[pl.ds(wid * rows, rows)], ids_vmem, sem
        ).wait()
        idx = ids_vmem[...]
        pltpu.make_async_copy(emb_hbm.at[idx], tile, sem).start()
        pltpu.make_async_copy(emb_hbm.at[idx], tile, sem).wait()
        pltpu.make_async_copy(
            tile, x_hbm.at[pl.ds(wid * rows, rows), :], sem
        ).start()
        pltpu.make_async_copy(
            tile, x_hbm.at[pl.ds(wid * rows, rows), :], sem
        ).wait()

    pl.run_scoped(
        scoped,
        pltpu.VMEM((16,), jnp.int32),
        pltpu.VMEM((16, 128), jnp.float32),
        pltpu.SemaphoreType.DMA,
    )


def _matmul_body(x_ref, w_ref, out_ref, xb_ref):
    seq = out_ref.shape[0]
    batch = out_ref.shape[2]

    @pl.when(pl.program_id(0) == 0)
    def _cast():
        xb_ref[...] = x_ref[...].astype(jnp.bfloat16)

    w = w_ref[...].astype(jnp.bfloat16)
    for s in range(seq):
        xs = xb_ref[s * batch : (s + 1) * batch, :]
        out_ref[s, :, :] = jax.lax.dot_general(
            w,
            xs,
            dimension_numbers=(((1,), (1,)), ((), ())),
            preferred_element_type=jnp.float32,
        )


def kernel(input_ids, embed_weight, lm_head_weight):
    batch, seq = input_ids.shape
    n_tok = batch * seq
    vocab, hidden = embed_weight.shape
    ids = input_ids.T.reshape(n_tok).astype(jnp.int32)

    sc_gather = pl.kernel(
        _sc_gather_body,
        out_type=jax.ShapeDtypeStruct((n_tok, hidden), jnp.float32),
        mesh=plsc.VectorSubcoreMesh(core_axis_name="c", subcore_axis_name="s"),
    )
    x = sc_gather(ids, embed_weight)

    n_tiles = pl.cdiv(vocab, _VOCAB_TILE)
    logits_svb = pl.pallas_call(
        _matmul_body,
        grid=(n_tiles,),
        in_specs=[
            pl.BlockSpec((n_tok, hidden), lambda j: (0, 0)),
            pl.BlockSpec((_VOCAB_TILE, hidden), lambda j: (j, 0)),
        ],
        out_specs=pl.BlockSpec((seq, _VOCAB_TILE, batch), lambda j: (0, j, 0)),
        scratch_shapes=[pltpu.VMEM((n_tok, hidden), jnp.bfloat16)],
        out_shape=jax.ShapeDtypeStruct((seq, vocab, batch), jnp.float32),
    )(x, lm_head_weight)

    return jnp.transpose(logits_svb, (2, 0, 1))


# speedup vs baseline: 1.0139x; 1.0139x over previous
import functools

import jax
import jax.numpy as jnp
from jax.experimental import pallas as pl
from jax.experimental.pallas import tpu as pltpu
from jax.experimental.pallas import tpu_sc as plsc

_VOCAB_TILE = 8192


def _sc_gather_body(ids_hbm, emb_hbm, x_hbm):
    num_cores = jax.lax.axis_size("c")
    c = jax.lax.axis_index("c")
    s = jax.lax.axis_index("s")
    wid = s * num_cores + c
    n_tok = x_hbm.shape[0]
    n_workers = num_cores * jax.lax.axis_size("s")
    rows = n_tok // n_workers

    def scoped(ids_vmem, tile, sem):
        pltpu.make_async_copy(
            ids_hbm.at[pl.ds(wid * rows, rows)], ids_vmem, sem
        ).start()
        pltpu.make_async_copy(
            ids_hbm.at[pl.ds(wid * rows, rows)], ids_vmem, sem
        ).wait()
        idx = ids_vmem[...]
        pltpu.make_async_copy(emb_hbm.at[idx], tile, sem).start()
        pltpu.make_async_copy(emb_hbm.at[idx], tile, sem).wait()
        pltpu.make_async_copy(
            tile, x_hbm.at[pl.ds(wid * rows, rows), :], sem
        ).start()
        pltpu.make_async_copy(
            tile, x_hbm.at[pl.ds(wid * rows, rows), :], sem
        ).wait()

    pl.run_scoped(
        scoped,
        pltpu.VMEM((16,), jnp.int32),
        pltpu.VMEM((16, 128), jnp.float32),
        pltpu.SemaphoreType.DMA,
    )


def _matmul_body(x_ref, w_ref, out_ref, xb_ref):
    seq = out_ref.shape[0]
    batch = out_ref.shape[2]

    @pl.when(pl.program_id(0) == 0)
    def _cast():
        xb_ref[...] = x_ref[...].astype(jnp.bfloat16)

    w = w_ref[...].astype(jnp.bfloat16)
    for s in range(seq):
        xs = xb_ref[s * batch : (s + 1) * batch, :]
        out_ref[s, :, :] = jax.lax.dot_general(
            w,
            xs,
            dimension_numbers=(((1,), (1,)), ((), ())),
            preferred_element_type=jnp.float32,
        )


def kernel(input_ids, embed_weight, lm_head_weight):
    batch, seq = input_ids.shape
    n_tok = batch * seq
    vocab, hidden = embed_weight.shape
    ids = input_ids.T.reshape(n_tok).astype(jnp.int32)

    sc_gather = pl.kernel(
        _sc_gather_body,
        out_type=jax.ShapeDtypeStruct((n_tok, hidden), jnp.float32),
        mesh=plsc.VectorSubcoreMesh(core_axis_name="c", subcore_axis_name="s"),
    )
    x = sc_gather(ids, embed_weight)

    n_tiles = pl.cdiv(vocab, _VOCAB_TILE)
    logits_svb = pl.pallas_call(
        _matmul_body,
        grid=(n_tiles,),
        in_specs=[
            pl.BlockSpec((n_tok, hidden), lambda j: (0, 0)),
            pl.BlockSpec((_VOCAB_TILE, hidden), lambda j: (j, 0)),
        ],
        out_specs=pl.BlockSpec((seq, _VOCAB_TILE, batch), lambda j: (0, j, 0)),
        scratch_shapes=[pltpu.VMEM((n_tok, hidden), jnp.bfloat16)],
        out_shape=jax.ShapeDtypeStruct((seq, vocab, batch), jnp.float32),
    )(x, lm_head_weight)

    return jnp.transpose(logits_svb, (2, 0, 1))
